# Initial kernel scaffold; baseline (speedup 1.0000x reference)
import jax
import jax.numpy as jnp
from jax.experimental import pallas as pl


def kernel(x, dw0, pw0, pb0, g0, be0, m0, v0, dw2, pw2, pb2, g2, be2, m2, v2, w1, b1, g1, be1, m1, v1, wh, bh, ww, bw):
    raise NotImplementedError("write your pallas kernel here")


# speedup vs baseline: 2.0173x; 2.0173x over previous
import jax
import jax.numpy as jnp
from jax import lax
from jax.experimental import pallas as pl
from jax.experimental.pallas import tpu as pltpu

_BN_EPS = 1e-5


def _fused_kernel(x_ref, wd1_ref, wp1_ref, bp1_ref, wd2_ref, wp2_ref, bp2_ref,
                  w1_ref, b1_ref, wh_ref, bh_ref, ww_ref, bw_ref,
                  o_ref, xp1_ref, xp2_ref):
    H, W, C = x_ref.shape
    Wp = xp1_ref.shape[1]

    def dw_conv(xp_ref, wd_ref):
        xs = [xp_ref[:, dx:dx + W, :] for dx in range(3)]
        wd = wd_ref[...]
        acc = jnp.zeros((H, W, C), jnp.float32)
        for dy in range(3):
            for dx in range(3):
                tap = wd[3 * dy + dx:3 * dy + dx + 1, :].reshape(1, 1, C)
                acc = acc + xs[dx][dy:dy + H, :, :] * tap
        return acc

    def pw_relu(acc, wp_ref, bp_ref):
        a2 = acc.reshape(H * W, C).astype(jnp.bfloat16)
        z = jnp.dot(a2, wp_ref[...], preferred_element_type=jnp.float32)
        return jnp.maximum(z + bp_ref[...], 0.0)

    def store_padded(xp_ref, val3d):
        xp_ref[0:1, :, :] = jnp.zeros((1, Wp, C), jnp.float32)
        xp_ref[H + 1:H + 2, :, :] = jnp.zeros((1, Wp, C), jnp.float32)
        xp_ref[:, 0:1, :] = jnp.zeros((H + 2, 1, C), jnp.float32)
        xp_ref[:, W + 1:W + 2, :] = jnp.zeros((H + 2, 1, C), jnp.float32)
        xp_ref[1:H + 1, 1:W + 1, :] = val3d

    store_padded(xp1_ref, x_ref[...])
    x1 = pw_relu(dw_conv(xp1_ref, wd1_ref), wp1_ref, bp1_ref)

    store_padded(xp2_ref, x1.reshape(H, W, C))
    x2 = pw_relu(dw_conv(xp2_ref, wd2_ref), wp2_ref, bp2_ref)
    x2_3d = x2.reshape(H, W, C)
    ph = jnp.mean(x2_3d, axis=1)
    pw_ = jnp.mean(x2_3d, axis=0)

    y = jnp.concatenate([ph, pw_], axis=0)
    y1 = jnp.dot(y, w1_ref[...], preferred_element_type=jnp.float32,
                 precision=lax.Precision.HIGHEST) + b1_ref[...]
    y2 = y1 * (jnp.clip(y1 + 3.0, 0.0, 6.0) * (1.0 / 6.0))
    a_h = jax.nn.sigmoid(
        jnp.dot(y2[0:H, :], wh_ref[...], preferred_element_type=jnp.float32,
                precision=lax.Precision.HIGHEST) + bh_ref[...])
    a_w = jax.nn.sigmoid(
        jnp.dot(y2[H:H + W, :], ww_ref[...], preferred_element_type=jnp.float32,
                precision=lax.Precision.HIGHEST) + bw_ref[...])

    o_ref[...] = (x2_3d * a_h[:, None, :] * a_w[None, :, :]).astype(o_ref.dtype)


def kernel(x, dw0, pw0, pb0, g0, be0, m0, v0, dw2, pw2, pb2, g2, be2, m2, v2,
           w1, b1, g1, be1, m1, v1, wh, bh, ww, bw):
    N, H, W, C = x.shape
    mip = w1.shape[1]

    s0 = g0 / jnp.sqrt(v0 + _BN_EPS)
    wp1f = (pw0 * s0[None, :]).astype(jnp.bfloat16)
    bp1f = (pb0 * s0 + be0 - m0 * s0).reshape(1, C).astype(jnp.float32)
    s2 = g2 / jnp.sqrt(v2 + _BN_EPS)
    wp2f = (pw2 * s2[None, :]).astype(jnp.bfloat16)
    bp2f = (pb2 * s2 + be2 - m2 * s2).reshape(1, C).astype(jnp.float32)
    s1 = g1 / jnp.sqrt(v1 + _BN_EPS)
    w1f = (w1 * s1[None, :]).astype(jnp.float32)
    b1f = ((b1 - m1) * s1 + be1).reshape(1, mip).astype(jnp.float32)

    wd1 = dw0.reshape(9, C).astype(jnp.float32)
    wd2 = dw2.reshape(9, C).astype(jnp.float32)
    bh2 = bh.reshape(1, C).astype(jnp.float32)
    bw2 = bw.reshape(1, C).astype(jnp.float32)

    full = lambda shape: pl.BlockSpec(shape, lambda n: tuple(0 for _ in shape))
    out = pl.pallas_call(
        _fused_kernel,
        out_shape=jax.ShapeDtypeStruct((N, H, W, C), x.dtype),
        grid=(N,),
        in_specs=[
            pl.BlockSpec((None, H, W, C), lambda n: (n, 0, 0, 0)),
            full((9, C)), full((C, C)), full((1, C)),
            full((9, C)), full((C, C)), full((1, C)),
            full((C, mip)), full((1, mip)),
            full((mip, C)), full((1, C)),
            full((mip, C)), full((1, C)),
        ],
        out_specs=pl.BlockSpec((None, H, W, C), lambda n: (n, 0, 0, 0)),
        scratch_shapes=[
            pltpu.VMEM((H + 2, W + 8, C), jnp.float32),
            pltpu.VMEM((H + 2, W + 8, C), jnp.float32),
        ],
        compiler_params=pltpu.CompilerParams(
            dimension_semantics=("parallel",),
            vmem_limit_bytes=48 * 1024 * 1024,
        ),
    )(x, wd1, wp1f, bp1f, wd2, wp2f, bp2f, w1f, b1f,
      wh.astype(jnp.float32), bh2, ww.astype(jnp.float32), bw2)

    return jnp.transpose(out, (0, 2, 1, 3))


# speedup vs baseline: 2.6690x; 1.3231x over previous
import jax
import jax.numpy as jnp
from jax import lax
from jax.experimental import pallas as pl
from jax.experimental.pallas import tpu as pltpu

_BN_EPS = 1e-5


def _fused_kernel(x_ref, wd1_ref, wp1_ref, bp1_ref, wd2_ref, wp2_ref, bp2_ref,
                  w1_ref, b1_ref, wh_ref, bh_ref, ww_ref, bw_ref,
                  o_ref, xp1_ref, xp2_ref):
    H, W, C = x_ref.shape
    Wp = xp1_ref.shape[1]

    def dw_conv(xp_ref, wd_ref):
        xs = [xp_ref[:, dx:dx + W, :] for dx in range(3)]
        wd = wd_ref[...]
        acc = jnp.zeros((H, W, C), jnp.float32)
        for dy in range(3):
            for dx in range(3):
                tap = wd[3 * dy + dx:3 * dy + dx + 1, :].reshape(1, 1, C)
                acc = acc + xs[dx][dy:dy + H, :, :] * tap
        return acc

    def pw_relu(acc, wp_ref, bp_ref):
        a2 = acc.reshape(H * W, C).astype(jnp.bfloat16)
        z = jnp.dot(a2, wp_ref[...], preferred_element_type=jnp.float32)
        return jnp.maximum(z + bp_ref[...], 0.0)

    def store_padded(xp_ref, val3d):
        xp_ref[0:1, :, :] = jnp.zeros((1, Wp, C), jnp.float32)
        xp_ref[H + 1:H + 2, :, :] = jnp.zeros((1, Wp, C), jnp.float32)
        xp_ref[:, 0:1, :] = jnp.zeros((H + 2, 1, C), jnp.float32)
        xp_ref[:, W + 1:W + 2, :] = jnp.zeros((H + 2, 1, C), jnp.float32)
        xp_ref[1:H + 1, 1:W + 1, :] = val3d

    store_padded(xp1_ref, x_ref[...])
    x1 = pw_relu(dw_conv(xp1_ref, wd1_ref), wp1_ref, bp1_ref)

    store_padded(xp2_ref, x1.reshape(H, W, C))
    x2 = pw_relu(dw_conv(xp2_ref, wd2_ref), wp2_ref, bp2_ref)
    x2_3d = x2.reshape(H, W, C)
    ph = jnp.mean(x2_3d, axis=1)
    pw_ = jnp.mean(x2_3d, axis=0)

    y = jnp.concatenate([ph, pw_], axis=0)
    y1 = jnp.dot(y, w1_ref[...], preferred_element_type=jnp.float32,
                 precision=lax.Precision.HIGHEST) + b1_ref[...]
    y2 = y1 * (jnp.clip(y1 + 3.0, 0.0, 6.0) * (1.0 / 6.0))
    a_h = jax.nn.sigmoid(
        jnp.dot(y2[0:H, :], wh_ref[...], preferred_element_type=jnp.float32,
                precision=lax.Precision.HIGHEST) + bh_ref[...])
    a_w = jax.nn.sigmoid(
        jnp.dot(y2[H:H + W, :], ww_ref[...], preferred_element_type=jnp.float32,
                precision=lax.Precision.HIGHEST) + bw_ref[...])

    res = x2_3d * a_h[:, None, :] * a_w[None, :, :]
    o_ref[...] = jnp.swapaxes(res, 0, 1).astype(o_ref.dtype)


def kernel(x, dw0, pw0, pb0, g0, be0, m0, v0, dw2, pw2, pb2, g2, be2, m2, v2,
           w1, b1, g1, be1, m1, v1, wh, bh, ww, bw):
    N, H, W, C = x.shape
    mip = w1.shape[1]

    s0 = g0 / jnp.sqrt(v0 + _BN_EPS)
    wp1f = (pw0 * s0[None, :]).astype(jnp.bfloat16)
    bp1f = (pb0 * s0 + be0 - m0 * s0).reshape(1, C).astype(jnp.float32)
    s2 = g2 / jnp.sqrt(v2 + _BN_EPS)
    wp2f = (pw2 * s2[None, :]).astype(jnp.bfloat16)
    bp2f = (pb2 * s2 + be2 - m2 * s2).reshape(1, C).astype(jnp.float32)
    s1 = g1 / jnp.sqrt(v1 + _BN_EPS)
    w1f = (w1 * s1[None, :]).astype(jnp.float32)
    b1f = ((b1 - m1) * s1 + be1).reshape(1, mip).astype(jnp.float32)

    wd1 = dw0.reshape(9, C).astype(jnp.float32)
    wd2 = dw2.reshape(9, C).astype(jnp.float32)
    bh2 = bh.reshape(1, C).astype(jnp.float32)
    bw2 = bw.reshape(1, C).astype(jnp.float32)

    full = lambda shape: pl.BlockSpec(shape, lambda n: tuple(0 for _ in shape))
    out = pl.pallas_call(
        _fused_kernel,
        out_shape=jax.ShapeDtypeStruct((N, W, H, C), x.dtype),
        grid=(N,),
        in_specs=[
            pl.BlockSpec((None, H, W, C), lambda n: (n, 0, 0, 0)),
            full((9, C)), full((C, C)), full((1, C)),
            full((9, C)), full((C, C)), full((1, C)),
            full((C, mip)), full((1, mip)),
            full((mip, C)), full((1, C)),
            full((mip, C)), full((1, C)),
        ],
        out_specs=pl.BlockSpec((None, W, H, C), lambda n: (n, 0, 0, 0)),
        scratch_shapes=[
            pltpu.VMEM((H + 2, W + 8, C), jnp.float32),
            pltpu.VMEM((H + 2, W + 8, C), jnp.float32),
        ],
        compiler_params=pltpu.CompilerParams(
            dimension_semantics=("parallel",),
            vmem_limit_bytes=48 * 1024 * 1024,
        ),
    )(x, wd1, wp1f, bp1f, wd2, wp2f, bp2f, w1f, b1f,
      wh.astype(jnp.float32), bh2, ww.astype(jnp.float32), bw2)

    return out


# speedup vs baseline: 3.4966x; 1.3101x over previous
import jax
import jax.numpy as jnp
from jax import lax
from jax.experimental import pallas as pl
from jax.experimental.pallas import tpu as pltpu

_BN_EPS = 1e-5


def _fused_kernel(x_ref, wd1_ref, wp1_ref, bp1_ref, wd2_ref, wp2_ref, bp2_ref,
                  w1_ref, b1_ref, wh_ref, bh_ref, ww_ref, bw_ref,
                  o_ref, xp1_ref, xp2_ref):
    H, W, C = x_ref.shape
    Wp = xp1_ref.shape[1]

    def dw_conv(xp_ref, wd_ref):
        xs = [xp_ref[:, dx:dx + W, :].astype(jnp.bfloat16) for dx in range(3)]
        wd = wd_ref[...]
        ps = []
        for dy in range(3):
            for dx in range(3):
                tap = wd[3 * dy + dx:3 * dy + dx + 1, :].reshape(1, 1, C)
                ps.append(xs[dx][dy:dy + H, :, :] * tap)
        while len(ps) > 1:
            nxt = [ps[i] + ps[i + 1] for i in range(0, len(ps) - 1, 2)]
            if len(ps) % 2:
                nxt.append(ps[-1])
            ps = nxt
        return ps[0]

    def pw_relu(acc, wp_ref, bp_ref):
        a2 = acc.reshape(H * W, C)
        z = jnp.dot(a2, wp_ref[...], preferred_element_type=jnp.float32)
        return jnp.maximum(z + bp_ref[...], 0.0)

    def store_padded(xp_ref, val3d):
        xp_ref[0:1, :, :] = jnp.zeros((1, Wp, C), jnp.float32)
        xp_ref[H + 1:H + 2, :, :] = jnp.zeros((1, Wp, C), jnp.float32)
        xp_ref[:, 0:1, :] = jnp.zeros((H + 2, 1, C), jnp.float32)
        xp_ref[:, W + 1:W + 2, :] = jnp.zeros((H + 2, 1, C), jnp.float32)
        xp_ref[1:H + 1, 1:W + 1, :] = val3d

    store_padded(xp1_ref, x_ref[...])
    x1 = pw_relu(dw_conv(xp1_ref, wd1_ref), wp1_ref, bp1_ref)

    store_padded(xp2_ref, x1.reshape(H, W, C))
    x2 = pw_relu(dw_conv(xp2_ref, wd2_ref), wp2_ref, bp2_ref)
    x2_3d = x2.reshape(H, W, C)
    ph = jnp.mean(x2_3d, axis=1)
    pw_ = jnp.mean(x2_3d, axis=0)

    y = jnp.concatenate([ph, pw_], axis=0)
    y1 = jnp.dot(y, w1_ref[...], preferred_element_type=jnp.float32,
                 precision=lax.Precision.HIGHEST) + b1_ref[...]
    y2 = y1 * (jnp.clip(y1 + 3.0, 0.0, 6.0) * (1.0 / 6.0))
    a_h = jax.nn.sigmoid(
        jnp.dot(y2[0:H, :], wh_ref[...], preferred_element_type=jnp.float32,
                precision=lax.Precision.HIGHEST) + bh_ref[...])
    a_w = jax.nn.sigmoid(
        jnp.dot(y2[H:H + W, :], ww_ref[...], preferred_element_type=jnp.float32,
                precision=lax.Precision.HIGHEST) + bw_ref[...])

    res = x2_3d * a_h[:, None, :] * a_w[None, :, :]
    o_ref[...] = jnp.swapaxes(res, 0, 1).astype(o_ref.dtype)


def kernel(x, dw0, pw0, pb0, g0, be0, m0, v0, dw2, pw2, pb2, g2, be2, m2, v2,
           w1, b1, g1, be1, m1, v1, wh, bh, ww, bw):
    N, H, W, C = x.shape
    mip = w1.shape[1]

    s0 = g0 / jnp.sqrt(v0 + _BN_EPS)
    wp1f = (pw0 * s0[None, :]).astype(jnp.bfloat16)
    bp1f = (pb0 * s0 + be0 - m0 * s0).reshape(1, C).astype(jnp.float32)
    s2 = g2 / jnp.sqrt(v2 + _BN_EPS)
    wp2f = (pw2 * s2[None, :]).astype(jnp.bfloat16)
    bp2f = (pb2 * s2 + be2 - m2 * s2).reshape(1, C).astype(jnp.float32)
    s1 = g1 / jnp.sqrt(v1 + _BN_EPS)
    w1f = (w1 * s1[None, :]).astype(jnp.float32)
    b1f = ((b1 - m1) * s1 + be1).reshape(1, mip).astype(jnp.float32)

    wd1 = dw0.reshape(9, C).astype(jnp.bfloat16)
    wd2 = dw2.reshape(9, C).astype(jnp.bfloat16)
    bh2 = bh.reshape(1, C).astype(jnp.float32)
    bw2 = bw.reshape(1, C).astype(jnp.float32)

    full = lambda shape: pl.BlockSpec(shape, lambda n: tuple(0 for _ in shape))
    out = pl.pallas_call(
        _fused_kernel,
        out_shape=jax.ShapeDtypeStruct((N, W, H, C), x.dtype),
        grid=(N,),
        in_specs=[
            pl.BlockSpec((None, H, W, C), lambda n: (n, 0, 0, 0)),
            full((9, C)), full((C, C)), full((1, C)),
            full((9, C)), full((C, C)), full((1, C)),
            full((C, mip)), full((1, mip)),
            full((mip, C)), full((1, C)),
            full((mip, C)), full((1, C)),
        ],
        out_specs=pl.BlockSpec((None, W, H, C), lambda n: (n, 0, 0, 0)),
        scratch_shapes=[
            pltpu.VMEM((H + 2, W + 8, C), jnp.float32),
            pltpu.VMEM((H + 2, W + 8, C), jnp.float32),
        ],
        compiler_params=pltpu.CompilerParams(
            dimension_semantics=("parallel",),
            vmem_limit_bytes=48 * 1024 * 1024,
        ),
    )(x, wd1, wp1f, bp1f, wd2, wp2f, bp2f, w1f, b1f,
      wh.astype(jnp.float32), bh2, ww.astype(jnp.float32), bw2)

    return out
